# Initial kernel scaffold; baseline (speedup 1.0000x reference)
import jax
import jax.numpy as jnp
from jax.experimental import pallas as pl


def kernel(x, edge_index, temp, bn_weight, bn_bias):
    raise NotImplementedError("write your pallas kernel here")


# speedup vs baseline: 13.0439x; 13.0439x over previous
import jax
import jax.numpy as jnp
from jax import lax
from jax.experimental import pallas as pl
from jax.experimental.pallas import tpu as pltpu
from jax.experimental.pallas import tpu_sc as plsc

N = 10000
E = 320000
D = 128
K = 10
EPS = 1e-5

NC = 2
NS = 16
L = 16
DH = D // NC
CH = DH // L
NP = 10240
RT = NP // NS
GH = NP - N
EP = 327680
W = 128
NWIN = EP // NS // W
WG = 8
NG = NWIN // WG
RC = 128
NCH = RT // RC


def _rsqrt(v):
    y = (v + 1.0) * 0.5
    for _ in range(15):
        y = (y + v / y) * 0.5
    return 1.0 / y


def _fill(ref, n, value):
    @pl.loop(0, n // L)
    def _(i):
        ref[pl.ds(i * L, L)] = jnp.full((L,), value, jnp.float32)


def _sc_body(x2, srcw, dstw, gam, bnw2, bnb2,
             out2,
             u_sp, acc_sp, degA_sp, degB_sp, stats_sp,
             sbuf, dbuf, bufY, bufH, nslice, tmpn,
             valA, valB, onesb, bnwv, bnbv, gamv,
             statv, statall,
             gsem, ssem, msem):
    cid = lax.axis_index("c")
    tid = lax.axis_index("s")
    r0 = tid * RT
    e0 = tid * NWIN
    nreal = jnp.minimum(RT, N - r0)
    myslice = pl.ds(r0, RT)

    out_hbm = out2.at[cid]
    x_hbm = x2.at[cid]

    pltpu.sync_copy(bnw2.at[cid], bnwv)
    pltpu.sync_copy(bnb2.at[cid], bnbv)
    pltpu.sync_copy(gam, gamv)
    _fill(onesb, W, 1.0)

    _fill(nslice, RT, 1.0)
    pltpu.sync_copy(nslice, degA_sp.at[myslice])
    plsc.subcore_barrier()

    @pl.loop(0, NG)
    def _(g):
        pltpu.sync_copy(dstw.at[pl.ds(e0 + g * WG, WG)], dbuf)

        @pl.loop(0, WG)
        def _(j):
            pltpu.sync_copy(onesb, degA_sp.at[dbuf.at[j]], add=True)
    plsc.subcore_barrier()

    pltpu.sync_copy(degA_sp.at[myslice], nslice)

    @pl.loop(0, RT // L)
    def _(i):
        d1 = _rsqrt(nslice[pl.ds(i * L, L)])
        nslice[pl.ds(i * L, L)] = d1
        tmpn[pl.ds(i * L, L)] = d1 * d1
    pltpu.sync_copy(nslice, degA_sp.at[myslice])
    pltpu.sync_copy(tmpn, degB_sp.at[myslice])
    plsc.subcore_barrier()

    @pl.loop(0, NG)
    def _(g):
        pltpu.sync_copy(srcw.at[pl.ds(e0 + g * WG, WG)], sbuf)
        pltpu.sync_copy(dstw.at[pl.ds(e0 + g * WG, WG)], dbuf)

        @pl.loop(0, WG)
        def _(j):
            pltpu.async_copy(degA_sp.at[sbuf.at[j]], valA, msem).wait()
            pltpu.async_copy(degA_sp.at[dbuf.at[j]], valB, msem).wait()

            @pl.loop(0, W // L)
            def _(i):
                valA[pl.ds(i * L, L)] = valA[pl.ds(i * L, L)] * valB[pl.ds(i * L, L)]
            pltpu.sync_copy(valA, degB_sp.at[sbuf.at[j]], add=True)
    plsc.subcore_barrier()

    pltpu.sync_copy(degB_sp.at[myslice], tmpn)

    @pl.loop(0, RT // L)
    def _(i):
        nslice[pl.ds(i * L, L)] = nslice[pl.ds(i * L, L)] * _rsqrt(tmpn[pl.ds(i * L, L)])
    gvec = gamv[pl.ds(0, L)]
    g0 = gvec[0]

    @pl.loop(0, NCH)
    def _(ch):
        rb = r0 + ch * RC
        pltpu.sync_copy(x_hbm.at[pl.ds(rb, RC)], bufH)

        @pl.loop(0, RC // L)
        def _(i):
            schunk = nslice[pl.ds(ch * RC + i * L, L)]
            for rr in range(L):
                r = i * L + rr
                sr = schunk[rr]
                for c in range(CH):
                    xr = bufH[r, pl.ds(c * L, L)]
                    bufY[r, pl.ds(c * L, L)] = sr * xr
                    bufH[r, pl.ds(c * L, L)] = g0 * xr
        pltpu.sync_copy(bufY, u_sp.at[pl.ds(rb, RC)])
        pltpu.sync_copy(bufY, acc_sp.at[pl.ds(rb, RC)])
        pltpu.sync_copy(bufH, out_hbm.at[pl.ds(rb, RC)])
    plsc.subcore_barrier()

    zero = jnp.zeros((L,), jnp.float32)

    @pl.loop(0, K)
    def _(k):
        @pl.loop(0, NG)
        def _(g):
            pltpu.sync_copy(srcw.at[pl.ds(e0 + g * WG, WG)], sbuf)
            pltpu.sync_copy(dstw.at[pl.ds(e0 + g * WG, WG)], dbuf)

            @pl.loop(0, WG)
            def _(j):
                pltpu.async_copy(u_sp.at[sbuf.at[j]], bufY, gsem.at[0]).wait()
                pltpu.async_copy(bufY, acc_sp.at[dbuf.at[j]], ssem.at[0], add=True).wait()
        plsc.subcore_barrier()

        def chunk_stats(ch, carry):
            sums = list(carry)
            rb = r0 + ch * RC
            pltpu.sync_copy(acc_sp.at[pl.ds(rb, RC)], bufY)
            nrows = jnp.clip(nreal - ch * RC, 0, RC)

            def stats_body(i, inner):
                isums = list(inner)
                schunk = nslice[pl.ds(ch * RC + i * L, L)]
                for rr in range(L):
                    r = i * L + rr
                    sr = schunk[rr]
                    for c in range(CH):
                        y = (-sr) * bufY[r, pl.ds(c * L, L)]
                        isums[c] = isums[c] + y
                        isums[CH + c] = isums[CH + c] + y * y
                return tuple(isums)

            return pl.loop(0, nrows // L, init_carry=tuple(sums))(stats_body)

        carry = pl.loop(0, NCH, init_carry=(zero,) * (2 * CH))(chunk_stats)
        for c in range(CH):
            statv[pl.ds(c * L, L)] = carry[c]
            statv[pl.ds(DH + c * L, L)] = carry[CH + c]
        pltpu.sync_copy(statv, stats_sp.at[tid])
        plsc.subcore_barrier()

        pltpu.sync_copy(stats_sp, statall)
        coeffs = []
        for c in range(CH):
            m = zero
            q = zero
            for t in range(NS):
                m = m + statall[t, pl.ds(c * L, L)]
                q = q + statall[t, pl.ds(DH + c * L, L)]
            m = m * (1.0 / N)
            q = q * (1.0 / N)
            inv = _rsqrt(q - m * m + EPS)
            gA = inv * bnwv[k, pl.ds(c * L, L)]
            gB = bnbv[k, pl.ds(c * L, L)] - m * gA
            coeffs.append((gA, gB))
        gvk = gamv[pl.ds(0, L)]
        gk = gvk.at[jnp.full((L,), k + 1, jnp.int32)].get(
            mode="promise_in_bounds")

        @pl.loop(0, NCH)
        def _(ch):
            rb = r0 + ch * RC
            pltpu.sync_copy(acc_sp.at[pl.ds(rb, RC)], bufY)
            pltpu.sync_copy(out_hbm.at[pl.ds(rb, RC)], bufH)

            @pl.loop(0, RC // L)
            def _(i):
                schunk = nslice[pl.ds(ch * RC + i * L, L)]
                for rr in range(L):
                    r = i * L + rr
                    sr = schunk[rr]
                    for c in range(CH):
                        gA, gB = coeffs[c]
                        h = ((-sr) * bufY[r, pl.ds(c * L, L)]) * gA + gB
                        bufH[r, pl.ds(c * L, L)] = bufH[r, pl.ds(c * L, L)] + gk * h
                        bufY[r, pl.ds(c * L, L)] = sr * h
            pltpu.sync_copy(bufH, out_hbm.at[pl.ds(rb, RC)])
            pltpu.sync_copy(bufY, u_sp.at[pl.ds(rb, RC)])
            pltpu.sync_copy(bufY, acc_sp.at[pl.ds(rb, RC)])
        plsc.subcore_barrier()


def _make_call():
    mesh = plsc.VectorSubcoreMesh(
        core_axis_name="c", subcore_axis_name="s",
        num_cores=NC, num_subcores=NS)
    f32 = jnp.float32
    return pl.kernel(
        _sc_body,
        out_type=jax.ShapeDtypeStruct((NC, NP, DH), f32),
        mesh=mesh,
        scratch_types=[
            pltpu.VMEM_SHARED((NP, DH), f32),
            pltpu.VMEM_SHARED((NP, DH), f32),
            pltpu.VMEM_SHARED((NP,), f32),
            pltpu.VMEM_SHARED((NP,), f32),
            pltpu.VMEM_SHARED((NS, 2 * DH), f32),
            pltpu.VMEM((WG, W), jnp.int32),
            pltpu.VMEM((WG, W), jnp.int32),
            pltpu.VMEM((RC, DH), f32),
            pltpu.VMEM((RC, DH), f32),
            pltpu.VMEM((RT,), f32),
            pltpu.VMEM((RT,), f32),
            pltpu.VMEM((W,), f32),
            pltpu.VMEM((W,), f32),
            pltpu.VMEM((W,), f32),
            pltpu.VMEM((K, DH), f32),
            pltpu.VMEM((K, DH), f32),
            pltpu.VMEM((L,), f32),
            pltpu.VMEM((2 * DH,), f32),
            pltpu.VMEM((NS, 2 * DH), f32),
            pltpu.SemaphoreType.DMA((4,)),
            pltpu.SemaphoreType.DMA((4,)),
            pltpu.SemaphoreType.DMA,
        ],
    )


_SC_CALL = _make_call()


def kernel(x, edge_index, temp, bn_weight, bn_bias):
    src = edge_index[0].astype(jnp.int32)
    dst = edge_index[1].astype(jnp.int32)
    npad = EP - E
    ghost = N + (jnp.arange(npad, dtype=jnp.int32) % GH)
    srcw = jnp.concatenate([src, ghost]).reshape(EP // W, W)
    dstw = jnp.concatenate([dst, ghost]).reshape(EP // W, W)
    xs = jnp.stack([x[:, :DH], x[:, DH:]])
    x2 = jnp.zeros((NC, NP, DH), jnp.float32).at[:, :N].set(xs)
    gam16 = jnp.zeros((L,), jnp.float32).at[:K + 1].set(temp / (K + 1))
    bnw2 = jnp.stack([bn_weight[:K, :DH], bn_weight[:K, DH:]])
    bnb2 = jnp.stack([bn_bias[:K, :DH], bn_bias[:K, DH:]])
    out2 = _SC_CALL(x2, srcw, dstw, gam16, bnw2, bnb2)
    return jnp.moveaxis(out2[:, :N], 0, 1).reshape(N, D)


# speedup vs baseline: 15.0179x; 1.1513x over previous
import jax
import jax.numpy as jnp
from jax import lax
from jax.experimental import pallas as pl
from jax.experimental.pallas import tpu as pltpu
from jax.experimental.pallas import tpu_sc as plsc

N = 10000
E = 320000
D = 128
K = 10
EPS = 1e-5

NC = 2
NS = 16
L = 16
DH = D // NC
CH = DH // L
NP = 10240
RT = NP // NS
GH = NP - N
EP = 327680
W = 128
NWIN = EP // NS // W
WG = 8
NG = NWIN // WG
RC = 128
NCH = RT // RC


def _rsqrt(v):
    y = (v + 1.0) * 0.5
    for _ in range(15):
        y = (y + v / y) * 0.5
    return 1.0 / y


def _fill(ref, n, value):
    @pl.loop(0, n // L)
    def _(i):
        ref[pl.ds(i * L, L)] = jnp.full((L,), value, jnp.float32)


def _sc_body(x2, srcw, dstw, gam, bnw2, bnb2,
             out2,
             u_sp, acc_sp, degA_sp, degB_sp, stats_sp,
             sbuf, dbuf, bufY, bufH, nslice, tmpn,
             valA, valB, onesb, bnwv, bnbv, gamv,
             statv, statall,
             gsem, ssem, msem):
    cid = lax.axis_index("c")
    tid = lax.axis_index("s")
    r0 = tid * RT
    e0 = tid * NWIN
    nreal = jnp.minimum(RT, N - r0)
    myslice = pl.ds(r0, RT)

    out_hbm = out2.at[cid]
    x_hbm = x2.at[cid]

    pltpu.sync_copy(bnw2.at[cid], bnwv)
    pltpu.sync_copy(bnb2.at[cid], bnbv)
    pltpu.sync_copy(gam, gamv)
    _fill(onesb, W, 1.0)

    _fill(nslice, RT, 1.0)
    pltpu.sync_copy(nslice, degA_sp.at[myslice])
    plsc.subcore_barrier()

    @pl.loop(0, NG)
    def _(g):
        pltpu.sync_copy(dstw.at[pl.ds(e0 + g * WG, WG)], dbuf)

        @pl.loop(0, WG)
        def _(j):
            pltpu.sync_copy(onesb, degA_sp.at[dbuf.at[j]], add=True)
    plsc.subcore_barrier()

    pltpu.sync_copy(degA_sp.at[myslice], nslice)

    @pl.loop(0, RT // L)
    def _(i):
        d1 = _rsqrt(nslice[pl.ds(i * L, L)])
        nslice[pl.ds(i * L, L)] = d1
        tmpn[pl.ds(i * L, L)] = d1 * d1
    pltpu.sync_copy(nslice, degA_sp.at[myslice])
    pltpu.sync_copy(tmpn, degB_sp.at[myslice])
    plsc.subcore_barrier()

    @pl.loop(0, NG)
    def _(g):
        pltpu.sync_copy(srcw.at[pl.ds(e0 + g * WG, WG)], sbuf)
        pltpu.sync_copy(dstw.at[pl.ds(e0 + g * WG, WG)], dbuf)

        @pl.loop(0, WG)
        def _(j):
            pltpu.async_copy(degA_sp.at[sbuf.at[j]], valA, msem).wait()
            pltpu.async_copy(degA_sp.at[dbuf.at[j]], valB, msem).wait()

            @pl.loop(0, W // L)
            def _(i):
                valA[pl.ds(i * L, L)] = valA[pl.ds(i * L, L)] * valB[pl.ds(i * L, L)]
            pltpu.sync_copy(valA, degB_sp.at[sbuf.at[j]], add=True)
    plsc.subcore_barrier()

    pltpu.sync_copy(degB_sp.at[myslice], tmpn)

    @pl.loop(0, RT // L)
    def _(i):
        nslice[pl.ds(i * L, L)] = nslice[pl.ds(i * L, L)] * _rsqrt(tmpn[pl.ds(i * L, L)])
    gvec = gamv[pl.ds(0, L)]
    g0 = gvec[0]

    @pl.loop(0, NCH)
    def _(ch):
        rb = r0 + ch * RC
        pltpu.sync_copy(x_hbm.at[pl.ds(rb, RC)], bufH)

        @pl.loop(0, RC // L)
        def _(i):
            schunk = nslice[pl.ds(ch * RC + i * L, L)]
            for rr in range(L):
                r = i * L + rr
                sr = schunk[rr]
                for c in range(CH):
                    xr = bufH[r, pl.ds(c * L, L)]
                    bufY[r, pl.ds(c * L, L)] = sr * xr
                    bufH[r, pl.ds(c * L, L)] = g0 * xr
        pltpu.sync_copy(bufY, u_sp.at[pl.ds(rb, RC)])
        pltpu.sync_copy(bufY, acc_sp.at[pl.ds(rb, RC)])
        pltpu.sync_copy(bufH, out_hbm.at[pl.ds(rb, RC)])
    plsc.subcore_barrier()

    zero = jnp.zeros((L,), jnp.float32)

    @pl.loop(0, K)
    def _(k):
        @pl.loop(0, NG)
        def _(g):
            pltpu.sync_copy(srcw.at[pl.ds(e0 + g * WG, WG)], sbuf)
            pltpu.sync_copy(dstw.at[pl.ds(e0 + g * WG, WG)], dbuf)
            bufs = (bufY, bufH)
            pltpu.async_copy(u_sp.at[sbuf.at[0]], bufs[0], gsem.at[0])
            for j in range(WG):
                b = j % 2
                nb = 1 - b
                if j + 1 < WG:
                    if j > 0:
                        pltpu.make_async_copy(
                            bufs[nb], acc_sp.at[dbuf.at[j - 1]], ssem.at[nb]).wait()
                    pltpu.async_copy(u_sp.at[sbuf.at[j + 1]], bufs[nb], gsem.at[nb])
                pltpu.make_async_copy(u_sp.at[sbuf.at[j]], bufs[b], gsem.at[b]).wait()
                pltpu.async_copy(bufs[b], acc_sp.at[dbuf.at[j]], ssem.at[b], add=True)
            for j in (WG - 2, WG - 1):
                b = j % 2
                pltpu.make_async_copy(
                    bufs[b], acc_sp.at[dbuf.at[j]], ssem.at[b]).wait()
        plsc.subcore_barrier()

        def chunk_stats(ch, carry):
            sums = list(carry)
            rb = r0 + ch * RC
            pltpu.sync_copy(acc_sp.at[pl.ds(rb, RC)], bufY)
            nrows = jnp.clip(nreal - ch * RC, 0, RC)

            def stats_body(i, inner):
                isums = list(inner)
                schunk = nslice[pl.ds(ch * RC + i * L, L)]
                for rr in range(L):
                    r = i * L + rr
                    sr = schunk[rr]
                    for c in range(CH):
                        y = (-sr) * bufY[r, pl.ds(c * L, L)]
                        isums[c] = isums[c] + y
                        isums[CH + c] = isums[CH + c] + y * y
                return tuple(isums)

            return pl.loop(0, nrows // L, init_carry=tuple(sums))(stats_body)

        carry = pl.loop(0, NCH, init_carry=(zero,) * (2 * CH))(chunk_stats)
        for c in range(CH):
            statv[pl.ds(c * L, L)] = carry[c]
            statv[pl.ds(DH + c * L, L)] = carry[CH + c]
        pltpu.sync_copy(statv, stats_sp.at[tid])
        plsc.subcore_barrier()

        pltpu.sync_copy(stats_sp, statall)
        coeffs = []
        for c in range(CH):
            m = zero
            q = zero
            for t in range(NS):
                m = m + statall[t, pl.ds(c * L, L)]
                q = q + statall[t, pl.ds(DH + c * L, L)]
            m = m * (1.0 / N)
            q = q * (1.0 / N)
            inv = _rsqrt(q - m * m + EPS)
            gA = inv * bnwv[k, pl.ds(c * L, L)]
            gB = bnbv[k, pl.ds(c * L, L)] - m * gA
            coeffs.append((gA, gB))
        gvk = gamv[pl.ds(0, L)]
        gk = gvk.at[jnp.full((L,), k + 1, jnp.int32)].get(
            mode="promise_in_bounds")

        @pl.loop(0, NCH)
        def _(ch):
            rb = r0 + ch * RC
            pltpu.sync_copy(acc_sp.at[pl.ds(rb, RC)], bufY)
            pltpu.sync_copy(out_hbm.at[pl.ds(rb, RC)], bufH)

            @pl.loop(0, RC // L)
            def _(i):
                schunk = nslice[pl.ds(ch * RC + i * L, L)]
                for rr in range(L):
                    r = i * L + rr
                    sr = schunk[rr]
                    for c in range(CH):
                        gA, gB = coeffs[c]
                        h = ((-sr) * bufY[r, pl.ds(c * L, L)]) * gA + gB
                        bufH[r, pl.ds(c * L, L)] = bufH[r, pl.ds(c * L, L)] + gk * h
                        bufY[r, pl.ds(c * L, L)] = sr * h
            pltpu.sync_copy(bufH, out_hbm.at[pl.ds(rb, RC)])
            pltpu.sync_copy(bufY, u_sp.at[pl.ds(rb, RC)])
            pltpu.sync_copy(bufY, acc_sp.at[pl.ds(rb, RC)])
        plsc.subcore_barrier()


def _make_call():
    mesh = plsc.VectorSubcoreMesh(
        core_axis_name="c", subcore_axis_name="s",
        num_cores=NC, num_subcores=NS)
    f32 = jnp.float32
    return pl.kernel(
        _sc_body,
        out_type=jax.ShapeDtypeStruct((NC, NP, DH), f32),
        mesh=mesh,
        scratch_types=[
            pltpu.VMEM_SHARED((NP, DH), f32),
            pltpu.VMEM_SHARED((NP, DH), f32),
            pltpu.VMEM_SHARED((NP,), f32),
            pltpu.VMEM_SHARED((NP,), f32),
            pltpu.VMEM_SHARED((NS, 2 * DH), f32),
            pltpu.VMEM((WG, W), jnp.int32),
            pltpu.VMEM((WG, W), jnp.int32),
            pltpu.VMEM((RC, DH), f32),
            pltpu.VMEM((RC, DH), f32),
            pltpu.VMEM((RT,), f32),
            pltpu.VMEM((RT,), f32),
            pltpu.VMEM((W,), f32),
            pltpu.VMEM((W,), f32),
            pltpu.VMEM((W,), f32),
            pltpu.VMEM((K, DH), f32),
            pltpu.VMEM((K, DH), f32),
            pltpu.VMEM((L,), f32),
            pltpu.VMEM((2 * DH,), f32),
            pltpu.VMEM((NS, 2 * DH), f32),
            pltpu.SemaphoreType.DMA((4,)),
            pltpu.SemaphoreType.DMA((4,)),
            pltpu.SemaphoreType.DMA,
        ],
    )


_SC_CALL = _make_call()


def kernel(x, edge_index, temp, bn_weight, bn_bias):
    src = edge_index[0].astype(jnp.int32)
    dst = edge_index[1].astype(jnp.int32)
    npad = EP - E
    ghost = N + (jnp.arange(npad, dtype=jnp.int32) % GH)
    srcw = jnp.concatenate([src, ghost]).reshape(EP // W, W)
    dstw = jnp.concatenate([dst, ghost]).reshape(EP // W, W)
    xs = jnp.stack([x[:, :DH], x[:, DH:]])
    x2 = jnp.zeros((NC, NP, DH), jnp.float32).at[:, :N].set(xs)
    gam16 = jnp.zeros((L,), jnp.float32).at[:K + 1].set(temp / (K + 1))
    bnw2 = jnp.stack([bn_weight[:K, :DH], bn_weight[:K, DH:]])
    bnb2 = jnp.stack([bn_bias[:K, :DH], bn_bias[:K, DH:]])
    out2 = _SC_CALL(x2, srcw, dstw, gam16, bnw2, bnb2)
    return jnp.moveaxis(out2[:, :N], 0, 1).reshape(N, D)


# speedup vs baseline: 22.0607x; 1.4690x over previous
import jax
import jax.numpy as jnp
from jax import lax
from jax.experimental import pallas as pl
from jax.experimental.pallas import tpu as pltpu
from jax.experimental.pallas import tpu_sc as plsc

N = 10000
E = 320000
D = 128
K = 10
EPS = 1e-5

NC = 2
NS = 16
L = 16
DH = D // NC
CH = DH // L
NP = 10240
RT = NP // NS
GH = NP - N
EP = 327680
W = 128
NWIN = EP // NS // W
WG = 16
NG = NWIN // WG
RC = 128
NCH = RT // RC


def _rsqrt(v):
    y = (v + 1.0) * 0.5
    for _ in range(15):
        y = (y + v / y) * 0.5
    return 1.0 / y


def _fill(ref, n, value):
    @pl.loop(0, n // L)
    def _(i):
        ref[pl.ds(i * L, L)] = jnp.full((L,), value, jnp.float32)


def _sc_body(x2, srcw, dstw, gam, bnw2, bnb2,
             out2, u_scr,
             acc_sp, degA_sp, degB_sp, stats_sp,
             sidx, didx, bufY, bufH, nslice, tmpn,
             valA, valB, onesb, bnwv, bnbv, gamv,
             statv, statall,
             gsem, ssem, msem):
    cid = lax.axis_index("c")
    tid = lax.axis_index("s")
    r0 = tid * RT
    e0 = tid * NWIN
    nreal = jnp.minimum(RT, N - r0)
    myslice = pl.ds(r0, RT)

    out_hbm = out2.at[cid]
    u_hbm = u_scr.at[cid]
    x_hbm = x2.at[cid]

    pltpu.sync_copy(srcw.at[pl.ds(e0, NWIN)], sidx)
    pltpu.sync_copy(dstw.at[pl.ds(e0, NWIN)], didx)
    pltpu.sync_copy(bnw2.at[cid], bnwv)
    pltpu.sync_copy(bnb2.at[cid], bnbv)
    pltpu.sync_copy(gam, gamv)
    _fill(onesb, W, 1.0)

    _fill(nslice, RT, 1.0)
    pltpu.sync_copy(nslice, degA_sp.at[myslice])
    plsc.subcore_barrier()

    @pl.loop(0, NWIN)
    def _(j):
        pltpu.sync_copy(onesb, degA_sp.at[didx.at[j]], add=True)
    plsc.subcore_barrier()

    pltpu.sync_copy(degA_sp.at[myslice], nslice)

    @pl.loop(0, RT // L)
    def _(i):
        d1 = _rsqrt(nslice[pl.ds(i * L, L)])
        nslice[pl.ds(i * L, L)] = d1
        tmpn[pl.ds(i * L, L)] = d1 * d1
    pltpu.sync_copy(nslice, degA_sp.at[myslice])
    pltpu.sync_copy(tmpn, degB_sp.at[myslice])
    plsc.subcore_barrier()

    @pl.loop(0, NWIN)
    def _(j):
        pltpu.async_copy(degA_sp.at[sidx.at[j]], valA, msem).wait()
        pltpu.async_copy(degA_sp.at[didx.at[j]], valB, msem).wait()

        @pl.loop(0, W // L)
        def _(i):
            valA[pl.ds(i * L, L)] = valA[pl.ds(i * L, L)] * valB[pl.ds(i * L, L)]
        pltpu.sync_copy(valA, degB_sp.at[sidx.at[j]], add=True)
    plsc.subcore_barrier()

    pltpu.sync_copy(degB_sp.at[myslice], tmpn)
    pltpu.sync_copy(degA_sp.at[myslice], nslice)

    @pl.loop(0, RT // L)
    def _(i):
        nslice[pl.ds(i * L, L)] = nslice[pl.ds(i * L, L)] * _rsqrt(tmpn[pl.ds(i * L, L)])
    gvec = gamv[pl.ds(0, L)]
    g0 = gvec[0]

    @pl.loop(0, NCH)
    def _(ch):
        rb = r0 + ch * RC
        pltpu.sync_copy(x_hbm.at[pl.ds(rb, RC)], bufH)

        @pl.loop(0, RC // L)
        def _(i):
            schunk = nslice[pl.ds(ch * RC + i * L, L)]
            for rr in range(L):
                r = i * L + rr
                sr = schunk[rr]
                for c in range(CH):
                    xr = bufH[r, pl.ds(c * L, L)]
                    bufY[r, pl.ds(c * L, L)] = sr * xr
                    bufH[r, pl.ds(c * L, L)] = g0 * xr
        pltpu.sync_copy(bufY, u_hbm.at[pl.ds(rb, RC)])
        pltpu.sync_copy(bufY, acc_sp.at[pl.ds(rb, RC)])
        pltpu.sync_copy(bufH, out_hbm.at[pl.ds(rb, RC)])
    plsc.subcore_barrier()

    zero = jnp.zeros((L,), jnp.float32)

    @pl.loop(0, K)
    def _(k):
        @pl.loop(0, NG)
        def _(g):
            j0 = g * WG
            bufs = (bufY.at[pl.ds(0, W)], bufH.at[pl.ds(0, W)])
            pltpu.async_copy(u_hbm.at[sidx.at[j0]], bufs[0], gsem.at[0])
            for jj in range(WG):
                b = jj % 2
                nb = 1 - b
                if jj + 1 < WG:
                    if jj > 0:
                        pltpu.make_async_copy(
                            bufs[nb], acc_sp.at[didx.at[j0 + jj - 1]],
                            ssem.at[nb]).wait()
                    pltpu.async_copy(
                        u_hbm.at[sidx.at[j0 + jj + 1]], bufs[nb], gsem.at[nb])
                pltpu.make_async_copy(
                    u_hbm.at[sidx.at[j0 + jj]], bufs[b], gsem.at[b]).wait()
                pltpu.async_copy(
                    bufs[b], acc_sp.at[didx.at[j0 + jj]], ssem.at[b], add=True)
            for jj in (WG - 2, WG - 1):
                b = jj % 2
                pltpu.make_async_copy(
                    bufs[b], acc_sp.at[didx.at[j0 + jj]], ssem.at[b]).wait()
        plsc.subcore_barrier()

        def chunk_stats(ch, carry):
            sums = list(carry)
            rb = r0 + ch * RC
            pltpu.sync_copy(acc_sp.at[pl.ds(rb, RC)], bufY)
            nrows = jnp.clip(nreal - ch * RC, 0, RC)

            def stats_body(i, inner):
                isums = list(inner)
                schunk = nslice[pl.ds(ch * RC + i * L, L)]
                for rr in range(L):
                    r = i * L + rr
                    sr = schunk[rr]
                    for c in range(CH):
                        y = (-sr) * bufY[r, pl.ds(c * L, L)]
                        isums[c] = isums[c] + y
                        isums[CH + c] = isums[CH + c] + y * y
                return tuple(isums)

            return pl.loop(0, nrows // L, init_carry=tuple(sums))(stats_body)

        carry = pl.loop(0, NCH, init_carry=(zero,) * (2 * CH))(chunk_stats)
        for c in range(CH):
            statv[pl.ds(c * L, L)] = carry[c]
            statv[pl.ds(DH + c * L, L)] = carry[CH + c]
        pltpu.sync_copy(statv, stats_sp.at[tid])
        plsc.subcore_barrier()

        pltpu.sync_copy(stats_sp, statall)
        coeffs = []
        for c in range(CH):
            m = zero
            q = zero
            for t in range(NS):
                m = m + statall[t, pl.ds(c * L, L)]
                q = q + statall[t, pl.ds(DH + c * L, L)]
            m = m * (1.0 / N)
            q = q * (1.0 / N)
            inv = _rsqrt(q - m * m + EPS)
            gA = inv * bnwv[k, pl.ds(c * L, L)]
            gB = bnbv[k, pl.ds(c * L, L)] - m * gA
            coeffs.append((gA, gB))
        gvk = gamv[pl.ds(0, L)]
        gk = gvk.at[jnp.full((L,), k + 1, jnp.int32)].get(
            mode="promise_in_bounds")

        @pl.loop(0, NCH)
        def _(ch):
            rb = r0 + ch * RC
            pltpu.sync_copy(acc_sp.at[pl.ds(rb, RC)], bufY)
            pltpu.sync_copy(out_hbm.at[pl.ds(rb, RC)], bufH)

            @pl.loop(0, RC // L)
            def _(i):
                schunk = nslice[pl.ds(ch * RC + i * L, L)]
                for rr in range(L):
                    r = i * L + rr
                    sr = schunk[rr]
                    for c in range(CH):
                        gA, gB = coeffs[c]
                        h = ((-sr) * bufY[r, pl.ds(c * L, L)]) * gA + gB
                        bufH[r, pl.ds(c * L, L)] = bufH[r, pl.ds(c * L, L)] + gk * h
                        bufY[r, pl.ds(c * L, L)] = sr * h
            pltpu.sync_copy(bufH, out_hbm.at[pl.ds(rb, RC)])
            pltpu.sync_copy(bufY, u_hbm.at[pl.ds(rb, RC)])
            pltpu.sync_copy(bufY, acc_sp.at[pl.ds(rb, RC)])
        plsc.subcore_barrier()


def _make_call():
    mesh = plsc.VectorSubcoreMesh(
        core_axis_name="c", subcore_axis_name="s",
        num_cores=NC, num_subcores=NS)
    f32 = jnp.float32
    return pl.kernel(
        _sc_body,
        out_type=(
            jax.ShapeDtypeStruct((NC, NP, DH), f32),
            jax.ShapeDtypeStruct((NC, NP, DH), f32),
        ),
        mesh=mesh,
        compiler_params=pltpu.CompilerParams(use_tc_tiling_on_sc=False),
        scratch_types=[
            pltpu.VMEM_SHARED((NP, DH), f32),
            pltpu.VMEM_SHARED((NP,), f32),
            pltpu.VMEM_SHARED((NP,), f32),
            pltpu.VMEM_SHARED((NS, 2 * DH), f32),
            pltpu.VMEM((NWIN, W), jnp.int32),
            pltpu.VMEM((NWIN, W), jnp.int32),
            pltpu.VMEM((RC, DH), f32),
            pltpu.VMEM((RC, DH), f32),
            pltpu.VMEM((RT,), f32),
            pltpu.VMEM((RT,), f32),
            pltpu.VMEM((W,), f32),
            pltpu.VMEM((W,), f32),
            pltpu.VMEM((W,), f32),
            pltpu.VMEM((K, DH), f32),
            pltpu.VMEM((K, DH), f32),
            pltpu.VMEM((L,), f32),
            pltpu.VMEM((2 * DH,), f32),
            pltpu.VMEM((NS, 2 * DH), f32),
            pltpu.SemaphoreType.DMA((4,)),
            pltpu.SemaphoreType.DMA((4,)),
            pltpu.SemaphoreType.DMA,
        ],
    )


_SC_CALL = _make_call()


def kernel(x, edge_index, temp, bn_weight, bn_bias):
    src = edge_index[0].astype(jnp.int32)
    dst = edge_index[1].astype(jnp.int32)
    npad = EP - E
    ghost = N + (jnp.arange(npad, dtype=jnp.int32) % GH)
    srcw = jnp.concatenate([src, ghost]).reshape(EP // W, W)
    dstw = jnp.concatenate([dst, ghost]).reshape(EP // W, W)
    xs = jnp.stack([x[:, :DH], x[:, DH:]])
    x2 = jnp.zeros((NC, NP, DH), jnp.float32).at[:, :N].set(xs)
    gam16 = jnp.zeros((L,), jnp.float32).at[:K + 1].set(temp / (K + 1))
    bnw2 = jnp.stack([bn_weight[:K, :DH], bn_weight[:K, DH:]])
    bnb2 = jnp.stack([bn_bias[:K, :DH], bn_bias[:K, DH:]])
    out2, _ = _SC_CALL(x2, srcw, dstw, gam16, bnw2, bnb2)
    return jnp.moveaxis(out2[:, :N], 0, 1).reshape(N, D)


# speedup vs baseline: 26.0857x; 1.1825x over previous
import jax
import jax.numpy as jnp
from jax import lax
from jax.experimental import pallas as pl
from jax.experimental.pallas import tpu as pltpu
from jax.experimental.pallas import tpu_sc as plsc

N = 10000
E = 320000
D = 128
K = 10
EPS = 1e-5

NC = 2
NS = 16
L = 16
DH = D // NC
CH = DH // L
NP = 10240
RT = NP // NS
GH = NP - N
EP = 327680
W = 128
NWIN = EP // NS // W
WG = 16
NG = NWIN // WG
RC = 128
NCH = RT // RC


def _rsqrt(v):
    y = (v + 1.0) * 0.5
    for _ in range(15):
        y = (y + v / y) * 0.5
    return 1.0 / y


def _fill(ref, n, value):
    @pl.loop(0, n // L)
    def _(i):
        ref[pl.ds(i * L, L)] = jnp.full((L,), value, jnp.float32)


def _sc_body(x2, srcw, dstw, gam, bnw2, bnb2,
             out2, u_scr,
             acc_sp, degA_sp, degB_sp, stats_sp,
             sidx, didx, bufY, bufH, wb2, wb3, nslice, tmpn,
             valA, valB, onesb, bnwv, bnbv, gamv,
             statv, statall,
             gsem, ssem, msem):
    cid = lax.axis_index("c")
    tid = lax.axis_index("s")
    r0 = tid * RT
    e0 = tid * NWIN
    nreal = jnp.minimum(RT, N - r0)
    myslice = pl.ds(r0, RT)

    out_hbm = out2.at[cid]
    u_hbm = u_scr.at[cid]
    x_hbm = x2.at[cid]

    pltpu.sync_copy(srcw.at[pl.ds(e0, NWIN)], sidx)
    pltpu.sync_copy(dstw.at[pl.ds(e0, NWIN)], didx)
    pltpu.sync_copy(bnw2.at[cid], bnwv)
    pltpu.sync_copy(bnb2.at[cid], bnbv)
    pltpu.sync_copy(gam, gamv)
    _fill(onesb, W, 1.0)

    _fill(nslice, RT, 1.0)
    pltpu.sync_copy(nslice, degA_sp.at[myslice])
    plsc.subcore_barrier()

    @pl.loop(0, NWIN)
    def _(j):
        pltpu.async_copy(onesb, degA_sp.at[didx.at[j]], msem, add=True)

    @pl.loop(0, NWIN)
    def _(j):
        pltpu.make_async_copy(onesb, degA_sp.at[didx.at[0]], msem).wait()
    plsc.subcore_barrier()

    pltpu.sync_copy(degA_sp.at[myslice], nslice)

    @pl.loop(0, RT // L)
    def _(i):
        nslice[pl.ds(i * L, L)] = _rsqrt(nslice[pl.ds(i * L, L)])
    pltpu.sync_copy(nslice, degA_sp.at[myslice])
    pltpu.sync_copy(nslice, degB_sp.at[myslice])
    plsc.subcore_barrier()

    pltpu.async_copy(degA_sp.at[didx.at[0]], valA, gsem.at[0])
    pltpu.async_copy(degA_sp.at[didx.at[1]], valB, gsem.at[1])

    @pl.loop(0, NWIN // 2)
    def _(i):
        j0 = 2 * i
        pltpu.make_async_copy(degA_sp.at[didx.at[j0]], valA, gsem.at[0]).wait()
        pltpu.async_copy(valA, degB_sp.at[sidx.at[j0]], ssem.at[0], add=True)
        pltpu.make_async_copy(degA_sp.at[didx.at[j0]], valB, gsem.at[1]).wait()
        pltpu.async_copy(valB, degB_sp.at[sidx.at[j0 + 1]], ssem.at[1], add=True)

        @pl.when(j0 + 2 < NWIN)
        def _():
            pltpu.make_async_copy(valA, degB_sp.at[sidx.at[j0]], ssem.at[0]).wait()
            pltpu.async_copy(degA_sp.at[didx.at[j0 + 2]], valA, gsem.at[0])

        @pl.when(j0 + 3 < NWIN)
        def _():
            pltpu.make_async_copy(valB, degB_sp.at[sidx.at[j0]], ssem.at[1]).wait()
            pltpu.async_copy(degA_sp.at[didx.at[j0 + 3]], valB, gsem.at[1])
    pltpu.make_async_copy(valA, degB_sp.at[sidx.at[0]], ssem.at[0]).wait()
    pltpu.make_async_copy(valB, degB_sp.at[sidx.at[0]], ssem.at[1]).wait()
    plsc.subcore_barrier()

    pltpu.sync_copy(degB_sp.at[myslice], tmpn)
    pltpu.sync_copy(degA_sp.at[myslice], nslice)

    @pl.loop(0, RT // L)
    def _(i):
        d1 = nslice[pl.ds(i * L, L)]
        nslice[pl.ds(i * L, L)] = d1 * _rsqrt(d1 * tmpn[pl.ds(i * L, L)])
    gvec = gamv[pl.ds(0, L)]
    g0 = gvec[0]

    @pl.loop(0, NCH)
    def _(ch):
        rb = r0 + ch * RC
        pltpu.sync_copy(x_hbm.at[pl.ds(rb, RC)], bufH)

        @pl.loop(0, RC // L)
        def _(i):
            schunk = nslice[pl.ds(ch * RC + i * L, L)]
            for rr in range(L):
                r = i * L + rr
                sr = schunk[rr]
                for c in range(CH):
                    xr = bufH[r, pl.ds(c * L, L)]
                    bufY[r, pl.ds(c * L, L)] = sr * xr
                    bufH[r, pl.ds(c * L, L)] = g0 * xr
        pltpu.sync_copy(bufY, u_hbm.at[pl.ds(rb, RC)])
        pltpu.sync_copy(bufY, acc_sp.at[pl.ds(rb, RC)])
        pltpu.sync_copy(bufH, out_hbm.at[pl.ds(rb, RC)])
    plsc.subcore_barrier()

    zero = jnp.zeros((L,), jnp.float32)

    @pl.loop(0, K)
    def _(k):
        wbufs = (bufY.at[pl.ds(0, W)], bufH.at[pl.ds(0, W)], wb2, wb3)
        for b in range(4):
            pltpu.async_copy(u_hbm.at[sidx.at[b]], wbufs[b], gsem.at[b])

        @pl.loop(0, NWIN // 4)
        def _(i):
            j0 = 4 * i
            for b in range(4):
                pltpu.make_async_copy(
                    u_hbm.at[sidx.at[j0]], wbufs[b], gsem.at[b]).wait()
                pltpu.async_copy(
                    wbufs[b], acc_sp.at[didx.at[j0 + b]], ssem.at[b], add=True)
            for b in range(4):
                @pl.when(j0 + 4 + b < NWIN)
                def _(b=b):
                    pltpu.make_async_copy(
                        wbufs[b], acc_sp.at[didx.at[j0]], ssem.at[b]).wait()
                    pltpu.async_copy(
                        u_hbm.at[sidx.at[j0 + 4 + b]], wbufs[b], gsem.at[b])
        for b in range(4):
            pltpu.make_async_copy(
                wbufs[b], acc_sp.at[didx.at[0]], ssem.at[b]).wait()
        plsc.subcore_barrier()

        def chunk_stats(ch, carry):
            sums = list(carry)
            rb = r0 + ch * RC
            pltpu.sync_copy(acc_sp.at[pl.ds(rb, RC)], bufY)
            nrows = jnp.clip(nreal - ch * RC, 0, RC)

            def stats_body(i, inner):
                isums = list(inner)
                schunk = nslice[pl.ds(ch * RC + i * L, L)]
                for rr in range(L):
                    r = i * L + rr
                    sr = schunk[rr]
                    for c in range(CH):
                        y = (-sr) * bufY[r, pl.ds(c * L, L)]
                        isums[c] = isums[c] + y
                        isums[CH + c] = isums[CH + c] + y * y
                return tuple(isums)

            return pl.loop(0, nrows // L, init_carry=tuple(sums))(stats_body)

        carry = pl.loop(0, NCH, init_carry=(zero,) * (2 * CH))(chunk_stats)
        for c in range(CH):
            statv[pl.ds(c * L, L)] = carry[c]
            statv[pl.ds(DH + c * L, L)] = carry[CH + c]
        pltpu.sync_copy(statv, stats_sp.at[tid])
        plsc.subcore_barrier()

        pltpu.sync_copy(stats_sp, statall)
        coeffs = []
        for c in range(CH):
            m = zero
            q = zero
            for t in range(NS):
                m = m + statall[t, pl.ds(c * L, L)]
                q = q + statall[t, pl.ds(DH + c * L, L)]
            m = m * (1.0 / N)
            q = q * (1.0 / N)
            inv = _rsqrt(q - m * m + EPS)
            gA = inv * bnwv[k, pl.ds(c * L, L)]
            gB = bnbv[k, pl.ds(c * L, L)] - m * gA
            coeffs.append((gA, gB))
        gvk = gamv[pl.ds(0, L)]
        gk = gvk.at[jnp.full((L,), k + 1, jnp.int32)].get(
            mode="promise_in_bounds")

        @pl.loop(0, NCH)
        def _(ch):
            rb = r0 + ch * RC
            pltpu.sync_copy(acc_sp.at[pl.ds(rb, RC)], bufY)
            pltpu.sync_copy(out_hbm.at[pl.ds(rb, RC)], bufH)

            @pl.loop(0, RC // L)
            def _(i):
                schunk = nslice[pl.ds(ch * RC + i * L, L)]
                for rr in range(L):
                    r = i * L + rr
                    sr = schunk[rr]
                    for c in range(CH):
                        gA, gB = coeffs[c]
                        h = ((-sr) * bufY[r, pl.ds(c * L, L)]) * gA + gB
                        bufH[r, pl.ds(c * L, L)] = bufH[r, pl.ds(c * L, L)] + gk * h
                        bufY[r, pl.ds(c * L, L)] = sr * h
            pltpu.sync_copy(bufH, out_hbm.at[pl.ds(rb, RC)])
            pltpu.sync_copy(bufY, u_hbm.at[pl.ds(rb, RC)])
            pltpu.sync_copy(bufY, acc_sp.at[pl.ds(rb, RC)])
        plsc.subcore_barrier()


def _make_call():
    mesh = plsc.VectorSubcoreMesh(
        core_axis_name="c", subcore_axis_name="s",
        num_cores=NC, num_subcores=NS)
    f32 = jnp.float32
    return pl.kernel(
        _sc_body,
        out_type=(
            jax.ShapeDtypeStruct((NC, NP, DH), f32),
            jax.ShapeDtypeStruct((NC, NP, DH), f32),
        ),
        mesh=mesh,
        compiler_params=pltpu.CompilerParams(use_tc_tiling_on_sc=False),
        scratch_types=[
            pltpu.VMEM_SHARED((NP, DH), f32),
            pltpu.VMEM_SHARED((NP,), f32),
            pltpu.VMEM_SHARED((NP,), f32),
            pltpu.VMEM_SHARED((NS, 2 * DH), f32),
            pltpu.VMEM((NWIN, W), jnp.int32),
            pltpu.VMEM((NWIN, W), jnp.int32),
            pltpu.VMEM((RC, DH), f32),
            pltpu.VMEM((RC, DH), f32),
            pltpu.VMEM((W, DH), f32),
            pltpu.VMEM((W, DH), f32),
            pltpu.VMEM((RT,), f32),
            pltpu.VMEM((RT,), f32),
            pltpu.VMEM((W,), f32),
            pltpu.VMEM((W,), f32),
            pltpu.VMEM((W,), f32),
            pltpu.VMEM((K, DH), f32),
            pltpu.VMEM((K, DH), f32),
            pltpu.VMEM((L,), f32),
            pltpu.VMEM((2 * DH,), f32),
            pltpu.VMEM((NS, 2 * DH), f32),
            pltpu.SemaphoreType.DMA((4,)),
            pltpu.SemaphoreType.DMA((4,)),
            pltpu.SemaphoreType.DMA,
        ],
    )


_SC_CALL = _make_call()


def kernel(x, edge_index, temp, bn_weight, bn_bias):
    src = edge_index[0].astype(jnp.int32)
    dst = edge_index[1].astype(jnp.int32)
    npad = EP - E
    ghost = N + (jnp.arange(npad, dtype=jnp.int32) % GH)
    srcw = jnp.concatenate([src, ghost]).reshape(EP // W, W)
    dstw = jnp.concatenate([dst, ghost]).reshape(EP // W, W)
    xs = jnp.stack([x[:, :DH], x[:, DH:]])
    x2 = jnp.zeros((NC, NP, DH), jnp.float32).at[:, :N].set(xs)
    gam16 = jnp.zeros((L,), jnp.float32).at[:K + 1].set(temp / (K + 1))
    bnw2 = jnp.stack([bn_weight[:K, :DH], bn_weight[:K, DH:]])
    bnb2 = jnp.stack([bn_bias[:K, :DH], bn_bias[:K, DH:]])
    out2, _ = _SC_CALL(x2, srcw, dstw, gam16, bnw2, bnb2)
    return jnp.moveaxis(out2[:, :N], 0, 1).reshape(N, D)


# speedup vs baseline: 28.0892x; 1.0768x over previous
import jax
import jax.numpy as jnp
from jax import lax
from jax.experimental import pallas as pl
from jax.experimental.pallas import tpu as pltpu
from jax.experimental.pallas import tpu_sc as plsc

N = 10000
E = 320000
D = 128
K = 10
EPS = 1e-5

NC = 2
NS = 16
L = 16
DH = D // NC
CH = DH // L
NP = 10240
RT = NP // NS
GH = NP - N
EP = 327680
W = 128
NWIN = EP // NS // W
WG = 16
NG = NWIN // WG
RC = 128
NCH = RT // RC


def _rsqrt(v):
    y = (v + 1.0) * 0.5
    for _ in range(15):
        y = (y + v / y) * 0.5
    return 1.0 / y


def _fill(ref, n, value):
    @pl.loop(0, n // L)
    def _(i):
        ref[pl.ds(i * L, L)] = jnp.full((L,), value, jnp.float32)


def _sc_body(x2, srcw, dstw, gam, bnw2, bnb2,
             out2, u_scr,
             acc_sp, degA_sp, degB_sp, stats_sp,
             sidx, didx, bufY, bufH, wb2, wb3, nslice, tmpn,
             valA, valB, onesb, bnwv, bnbv, gamv,
             statv, statall,
             gsem, ssem, osem, msem):
    cid = lax.axis_index("c")
    tid = lax.axis_index("s")
    r0 = tid * RT
    e0 = tid * NWIN
    nreal = jnp.minimum(RT, N - r0)
    myslice = pl.ds(r0, RT)

    out_hbm = out2.at[cid]
    u_hbm = u_scr.at[cid]
    x_hbm = x2.at[cid]

    pltpu.sync_copy(srcw.at[pl.ds(e0, NWIN)], sidx)
    pltpu.sync_copy(dstw.at[pl.ds(e0, NWIN)], didx)
    pltpu.sync_copy(bnw2.at[cid], bnwv)
    pltpu.sync_copy(bnb2.at[cid], bnbv)
    pltpu.sync_copy(gam, gamv)
    _fill(onesb, W, 1.0)

    _fill(nslice, RT, 1.0)
    pltpu.sync_copy(nslice, degA_sp.at[myslice])
    plsc.subcore_barrier()

    @pl.loop(0, NWIN)
    def _(j):
        pltpu.async_copy(onesb, degA_sp.at[didx.at[j]], msem, add=True)

    @pl.loop(0, NWIN)
    def _(j):
        pltpu.make_async_copy(onesb, degA_sp.at[didx.at[0]], msem).wait()
    plsc.subcore_barrier()

    pltpu.sync_copy(degA_sp.at[myslice], nslice)

    @pl.loop(0, RT // L)
    def _(i):
        nslice[pl.ds(i * L, L)] = _rsqrt(nslice[pl.ds(i * L, L)])
    pltpu.sync_copy(nslice, degA_sp.at[myslice])
    pltpu.sync_copy(nslice, degB_sp.at[myslice])
    plsc.subcore_barrier()

    pltpu.async_copy(degA_sp.at[didx.at[0]], valA, gsem.at[0])
    pltpu.async_copy(degA_sp.at[didx.at[1]], valB, gsem.at[1])

    @pl.loop(0, NWIN // 2)
    def _(i):
        j0 = 2 * i
        pltpu.make_async_copy(degA_sp.at[didx.at[j0]], valA, gsem.at[0]).wait()
        pltpu.async_copy(valA, degB_sp.at[sidx.at[j0]], ssem.at[0], add=True)
        pltpu.make_async_copy(degA_sp.at[didx.at[j0]], valB, gsem.at[1]).wait()
        pltpu.async_copy(valB, degB_sp.at[sidx.at[j0 + 1]], ssem.at[1], add=True)

        @pl.when(j0 + 2 < NWIN)
        def _():
            pltpu.make_async_copy(valA, degB_sp.at[sidx.at[j0]], ssem.at[0]).wait()
            pltpu.async_copy(degA_sp.at[didx.at[j0 + 2]], valA, gsem.at[0])

        @pl.when(j0 + 3 < NWIN)
        def _():
            pltpu.make_async_copy(valB, degB_sp.at[sidx.at[j0]], ssem.at[1]).wait()
            pltpu.async_copy(degA_sp.at[didx.at[j0 + 3]], valB, gsem.at[1])
    pltpu.make_async_copy(valA, degB_sp.at[sidx.at[0]], ssem.at[0]).wait()
    pltpu.make_async_copy(valB, degB_sp.at[sidx.at[0]], ssem.at[1]).wait()
    plsc.subcore_barrier()

    pltpu.sync_copy(degB_sp.at[myslice], tmpn)
    pltpu.sync_copy(degA_sp.at[myslice], nslice)

    @pl.loop(0, RT // L)
    def _(i):
        d1 = nslice[pl.ds(i * L, L)]
        nslice[pl.ds(i * L, L)] = d1 * _rsqrt(d1 * tmpn[pl.ds(i * L, L)])
    gvec = gamv[pl.ds(0, L)]
    g0 = gvec[0]

    @pl.loop(0, NCH)
    def _(ch):
        rb = r0 + ch * RC
        pltpu.sync_copy(x_hbm.at[pl.ds(rb, RC)], bufH)

        @pl.loop(0, RC // L)
        def _(i):
            schunk = nslice[pl.ds(ch * RC + i * L, L)]
            for rr in range(L):
                r = i * L + rr
                sr = schunk[rr]
                for c in range(CH):
                    xr = bufH[r, pl.ds(c * L, L)]
                    bufY[r, pl.ds(c * L, L)] = sr * xr
                    bufH[r, pl.ds(c * L, L)] = g0 * xr
        pltpu.sync_copy(bufY, u_hbm.at[pl.ds(rb, RC)])
        pltpu.sync_copy(bufY, acc_sp.at[pl.ds(rb, RC)])
        pltpu.sync_copy(bufH, out_hbm.at[pl.ds(rb, RC)])
    plsc.subcore_barrier()

    zero = jnp.zeros((L,), jnp.float32)

    @pl.loop(0, K)
    def _(k):
        wbufs = (bufY.at[pl.ds(0, W)], bufH.at[pl.ds(0, W)], wb2, wb3)
        for b in range(4):
            pltpu.async_copy(u_hbm.at[sidx.at[b]], wbufs[b], gsem.at[b])

        @pl.loop(0, NWIN // 4)
        def _(i):
            j0 = 4 * i
            for b in range(4):
                pltpu.make_async_copy(
                    u_hbm.at[sidx.at[j0]], wbufs[b], gsem.at[b]).wait()
                pltpu.async_copy(
                    wbufs[b], acc_sp.at[didx.at[j0 + b]], ssem.at[b], add=True)
            for b in range(4):
                @pl.when(j0 + 4 + b < NWIN)
                def _(b=b):
                    pltpu.make_async_copy(
                        wbufs[b], acc_sp.at[didx.at[j0]], ssem.at[b]).wait()
                    pltpu.async_copy(
                        u_hbm.at[sidx.at[j0 + 4 + b]], wbufs[b], gsem.at[b])
        for b in range(4):
            pltpu.make_async_copy(
                wbufs[b], acc_sp.at[didx.at[0]], ssem.at[b]).wait()
        plsc.subcore_barrier()

        sbufs = (bufY, wb2)
        pltpu.async_copy(acc_sp.at[pl.ds(r0, RC)], sbufs[0], gsem.at[0])
        carry = (zero,) * (2 * CH)
        for ch in range(NCH):
            cb = sbufs[ch % 2]
            pltpu.make_async_copy(
                acc_sp.at[pl.ds(r0, RC)], cb, gsem.at[ch % 2]).wait()
            if ch + 1 < NCH:
                pltpu.async_copy(
                    acc_sp.at[pl.ds(r0 + (ch + 1) * RC, RC)],
                    sbufs[(ch + 1) % 2], gsem.at[(ch + 1) % 2])
            nrows = jnp.clip(nreal - ch * RC, 0, RC)

            def stats_body(i, inner, ch=ch, cb=cb):
                isums = list(inner)
                schunk = nslice[pl.ds(ch * RC + i * L, L)]
                for rr in range(L):
                    r = i * L + rr
                    sr = schunk[rr]
                    for c in range(CH):
                        y = (-sr) * cb[r, pl.ds(c * L, L)]
                        isums[c] = isums[c] + y
                        isums[CH + c] = isums[CH + c] + y * y
                return tuple(isums)

            carry = pl.loop(0, nrows // L, init_carry=carry)(stats_body)
        for c in range(CH):
            statv[pl.ds(c * L, L)] = carry[c]
            statv[pl.ds(DH + c * L, L)] = carry[CH + c]
        pltpu.sync_copy(statv, stats_sp.at[tid])
        plsc.subcore_barrier()

        pltpu.sync_copy(stats_sp, statall)
        coeffs = []
        for c in range(CH):
            m = zero
            q = zero
            for t in range(NS):
                m = m + statall[t, pl.ds(c * L, L)]
                q = q + statall[t, pl.ds(DH + c * L, L)]
            m = m * (1.0 / N)
            q = q * (1.0 / N)
            inv = _rsqrt(q - m * m + EPS)
            gA = inv * bnwv[k, pl.ds(c * L, L)]
            gB = bnbv[k, pl.ds(c * L, L)] - m * gA
            coeffs.append((gA, gB))
        gvk = gamv[pl.ds(0, L)]
        gk = gvk.at[jnp.full((L,), k + 1, jnp.int32)].get(
            mode="promise_in_bounds")

        pairs = ((bufY, bufH), (wb2, wb3))
        pltpu.async_copy(acc_sp.at[pl.ds(r0, RC)], bufY, gsem.at[0])
        pltpu.async_copy(out_hbm.at[pl.ds(r0, RC)], bufH, gsem.at[2])
        for ch in range(NCH):
            p = ch % 2
            Yb, Hb = pairs[p]
            pltpu.make_async_copy(
                acc_sp.at[pl.ds(r0, RC)], Yb, gsem.at[p]).wait()
            pltpu.make_async_copy(
                out_hbm.at[pl.ds(r0, RC)], Hb, gsem.at[p + 2]).wait()
            if ch + 1 < NCH:
                np_ = (ch + 1) % 2
                Yn, Hn = pairs[np_]
                if ch >= 1:
                    pltpu.make_async_copy(
                        Yn, u_hbm.at[pl.ds(r0, RC)], ssem.at[np_]).wait()
                    pltpu.make_async_copy(
                        Yn, acc_sp.at[pl.ds(r0, RC)], ssem.at[np_ + 2]).wait()
                    pltpu.make_async_copy(
                        Hn, out_hbm.at[pl.ds(r0, RC)], osem.at[np_]).wait()
                pltpu.async_copy(
                    acc_sp.at[pl.ds(r0 + (ch + 1) * RC, RC)], Yn, gsem.at[np_])
                pltpu.async_copy(
                    out_hbm.at[pl.ds(r0 + (ch + 1) * RC, RC)], Hn,
                    gsem.at[np_ + 2])

            @pl.loop(0, RC // L)
            def _(i, ch=ch, Yb=Yb, Hb=Hb):
                schunk = nslice[pl.ds(ch * RC + i * L, L)]
                for rr in range(L):
                    r = i * L + rr
                    sr = schunk[rr]
                    for c in range(CH):
                        gA, gB = coeffs[c]
                        h = ((-sr) * Yb[r, pl.ds(c * L, L)]) * gA + gB
                        Hb[r, pl.ds(c * L, L)] = Hb[r, pl.ds(c * L, L)] + gk * h
                        Yb[r, pl.ds(c * L, L)] = sr * h
            rb = r0 + ch * RC
            pltpu.async_copy(Hb, out_hbm.at[pl.ds(rb, RC)], osem.at[p])
            pltpu.async_copy(Yb, u_hbm.at[pl.ds(rb, RC)], ssem.at[p])
            pltpu.async_copy(Yb, acc_sp.at[pl.ds(rb, RC)], ssem.at[p + 2])
        for ch in (NCH - 2, NCH - 1):
            p = ch % 2
            Yb, Hb = pairs[p]
            pltpu.make_async_copy(
                Yb, u_hbm.at[pl.ds(r0, RC)], ssem.at[p]).wait()
            pltpu.make_async_copy(
                Yb, acc_sp.at[pl.ds(r0, RC)], ssem.at[p + 2]).wait()
            pltpu.make_async_copy(
                Hb, out_hbm.at[pl.ds(r0, RC)], osem.at[p]).wait()
        plsc.subcore_barrier()


def _make_call():
    mesh = plsc.VectorSubcoreMesh(
        core_axis_name="c", subcore_axis_name="s",
        num_cores=NC, num_subcores=NS)
    f32 = jnp.float32
    return pl.kernel(
        _sc_body,
        out_type=(
            jax.ShapeDtypeStruct((NC, NP, DH), f32),
            jax.ShapeDtypeStruct((NC, NP, DH), f32),
        ),
        mesh=mesh,
        compiler_params=pltpu.CompilerParams(use_tc_tiling_on_sc=False),
        scratch_types=[
            pltpu.VMEM_SHARED((NP, DH), f32),
            pltpu.VMEM_SHARED((NP,), f32),
            pltpu.VMEM_SHARED((NP,), f32),
            pltpu.VMEM_SHARED((NS, 2 * DH), f32),
            pltpu.VMEM((NWIN, W), jnp.int32),
            pltpu.VMEM((NWIN, W), jnp.int32),
            pltpu.VMEM((RC, DH), f32),
            pltpu.VMEM((RC, DH), f32),
            pltpu.VMEM((W, DH), f32),
            pltpu.VMEM((W, DH), f32),
            pltpu.VMEM((RT,), f32),
            pltpu.VMEM((RT,), f32),
            pltpu.VMEM((W,), f32),
            pltpu.VMEM((W,), f32),
            pltpu.VMEM((W,), f32),
            pltpu.VMEM((K, DH), f32),
            pltpu.VMEM((K, DH), f32),
            pltpu.VMEM((L,), f32),
            pltpu.VMEM((2 * DH,), f32),
            pltpu.VMEM((NS, 2 * DH), f32),
            pltpu.SemaphoreType.DMA((4,)),
            pltpu.SemaphoreType.DMA((4,)),
            pltpu.SemaphoreType.DMA((2,)),
            pltpu.SemaphoreType.DMA,
        ],
    )


_SC_CALL = _make_call()


def kernel(x, edge_index, temp, bn_weight, bn_bias):
    src = edge_index[0].astype(jnp.int32)
    dst = edge_index[1].astype(jnp.int32)
    npad = EP - E
    ghost = N + (jnp.arange(npad, dtype=jnp.int32) % GH)
    srcw = jnp.concatenate([src, ghost]).reshape(EP // W, W)
    dstw = jnp.concatenate([dst, ghost]).reshape(EP // W, W)
    xs = jnp.stack([x[:, :DH], x[:, DH:]])
    x2 = jnp.zeros((NC, NP, DH), jnp.float32).at[:, :N].set(xs)
    gam16 = jnp.zeros((L,), jnp.float32).at[:K + 1].set(temp / (K + 1))
    bnw2 = jnp.stack([bn_weight[:K, :DH], bn_weight[:K, DH:]])
    bnb2 = jnp.stack([bn_bias[:K, :DH], bn_bias[:K, DH:]])
    out2, _ = _SC_CALL(x2, srcw, dstw, gam16, bnw2, bnb2)
    return jnp.moveaxis(out2[:, :N], 0, 1).reshape(N, D)


# speedup vs baseline: 28.1686x; 1.0028x over previous
import jax
import jax.numpy as jnp
from jax import lax
from jax.experimental import pallas as pl
from jax.experimental.pallas import tpu as pltpu
from jax.experimental.pallas import tpu_sc as plsc

N = 10000
E = 320000
D = 128
K = 10
EPS = 1e-5

NC = 2
NS = 16
L = 16
DH = D // NC
CH = DH // L
NP = 10240
RT = NP // NS
GH = NP - N
EP = 327680
W = 128
NWIN = EP // NS // W
WG = 16
NG = NWIN // WG
RC = 128
NCH = RT // RC


def _rsqrt(v):
    y = (v + 1.0) * 0.5
    for _ in range(15):
        y = (y + v / y) * 0.5
    return 1.0 / y


def _fill(ref, n, value):
    @pl.loop(0, n // L)
    def _(i):
        ref[pl.ds(i * L, L)] = jnp.full((L,), value, jnp.float32)


def _sc_body(x2, srcw, dstw, gam, bnw2, bnb2,
             out2, u_scr,
             acc_sp, degA_sp, degB_sp, stats_sp,
             sidx, didx, bufY, bufH, wb2, wb3, nslice, tmpn,
             valA, valB, onesb, bnwv, bnbv, gamv,
             statv, statall,
             gsem, ssem, osem, msem):
    cid = lax.axis_index("c")
    tid = lax.axis_index("s")
    r0 = tid * RT
    e0 = tid * NWIN
    nreal = jnp.minimum(RT, N - r0)
    myslice = pl.ds(r0, RT)

    out_hbm = out2.at[cid]
    u_hbm = u_scr.at[cid]
    x_hbm = x2.at[cid]

    pltpu.sync_copy(srcw.at[pl.ds(e0, NWIN)], sidx)
    pltpu.sync_copy(dstw.at[pl.ds(e0, NWIN)], didx)
    pltpu.sync_copy(bnw2.at[cid], bnwv)
    pltpu.sync_copy(bnb2.at[cid], bnbv)
    pltpu.sync_copy(gam, gamv)
    _fill(onesb, W, 1.0)

    _fill(nslice, RT, 1.0)
    pltpu.sync_copy(nslice, degA_sp.at[myslice])
    plsc.subcore_barrier()

    @pl.loop(0, NWIN)
    def _(j):
        pltpu.async_copy(onesb, degA_sp.at[didx.at[j]], msem, add=True)

    @pl.loop(0, NWIN)
    def _(j):
        pltpu.make_async_copy(onesb, degA_sp.at[didx.at[0]], msem).wait()
    plsc.subcore_barrier()

    pltpu.sync_copy(degA_sp.at[myslice], nslice)

    @pl.loop(0, RT // L)
    def _(i):
        nslice[pl.ds(i * L, L)] = _rsqrt(nslice[pl.ds(i * L, L)])
    pltpu.sync_copy(nslice, degA_sp.at[myslice])
    pltpu.sync_copy(nslice, degB_sp.at[myslice])
    plsc.subcore_barrier()

    pltpu.async_copy(degA_sp.at[didx.at[0]], valA, gsem.at[0])
    pltpu.async_copy(degA_sp.at[didx.at[1]], valB, gsem.at[1])

    @pl.loop(0, NWIN // 2)
    def _(i):
        j0 = 2 * i
        pltpu.make_async_copy(degA_sp.at[didx.at[j0]], valA, gsem.at[0]).wait()
        pltpu.async_copy(valA, degB_sp.at[sidx.at[j0]], ssem.at[0], add=True)
        pltpu.make_async_copy(degA_sp.at[didx.at[j0]], valB, gsem.at[1]).wait()
        pltpu.async_copy(valB, degB_sp.at[sidx.at[j0 + 1]], ssem.at[1], add=True)

        @pl.when(j0 + 2 < NWIN)
        def _():
            pltpu.make_async_copy(valA, degB_sp.at[sidx.at[j0]], ssem.at[0]).wait()
            pltpu.async_copy(degA_sp.at[didx.at[j0 + 2]], valA, gsem.at[0])

        @pl.when(j0 + 3 < NWIN)
        def _():
            pltpu.make_async_copy(valB, degB_sp.at[sidx.at[j0]], ssem.at[1]).wait()
            pltpu.async_copy(degA_sp.at[didx.at[j0 + 3]], valB, gsem.at[1])
    pltpu.make_async_copy(valA, degB_sp.at[sidx.at[0]], ssem.at[0]).wait()
    pltpu.make_async_copy(valB, degB_sp.at[sidx.at[0]], ssem.at[1]).wait()
    plsc.subcore_barrier()

    pltpu.sync_copy(degB_sp.at[myslice], tmpn)
    pltpu.sync_copy(degA_sp.at[myslice], nslice)

    @pl.loop(0, RT // L)
    def _(i):
        d1 = nslice[pl.ds(i * L, L)]
        nslice[pl.ds(i * L, L)] = d1 * _rsqrt(d1 * tmpn[pl.ds(i * L, L)])
    gvec = gamv[pl.ds(0, L)]
    g0 = gvec[0]

    @pl.loop(0, NCH)
    def _(ch):
        rb = r0 + ch * RC
        pltpu.sync_copy(x_hbm.at[pl.ds(rb, RC)], bufH)

        @pl.loop(0, RC // L)
        def _(i):
            schunk = nslice[pl.ds(ch * RC + i * L, L)]
            for rr in range(L):
                r = i * L + rr
                sr = schunk[rr]
                for c in range(CH):
                    xr = bufH[r, pl.ds(c * L, L)]
                    bufY[r, pl.ds(c * L, L)] = sr * xr
                    bufH[r, pl.ds(c * L, L)] = g0 * xr
        pltpu.sync_copy(bufY, u_hbm.at[pl.ds(rb, RC)])
        pltpu.sync_copy(bufY, acc_sp.at[pl.ds(rb, RC)])
        pltpu.sync_copy(bufH, out_hbm.at[pl.ds(rb, RC)])
    plsc.subcore_barrier()

    zero = jnp.zeros((L,), jnp.float32)

    @pl.loop(0, K)
    def _(k):
        wbufs = (bufY.at[pl.ds(0, W)], bufH.at[pl.ds(0, W)], wb2, wb3)
        for b in range(4):
            pltpu.async_copy(u_hbm.at[sidx.at[b]], wbufs[b], gsem.at[b])

        @pl.loop(0, NWIN // 4)
        def _(i):
            j0 = 4 * i
            for b in range(4):
                pltpu.make_async_copy(
                    u_hbm.at[sidx.at[j0]], wbufs[b], gsem.at[b]).wait()
                pltpu.async_copy(
                    wbufs[b], acc_sp.at[didx.at[j0 + b]], ssem.at[b], add=True)
            for b in range(4):
                @pl.when(j0 + 4 + b < NWIN)
                def _(b=b):
                    pltpu.make_async_copy(
                        wbufs[b], acc_sp.at[didx.at[j0]], ssem.at[b]).wait()
                    pltpu.async_copy(
                        u_hbm.at[sidx.at[j0 + 4 + b]], wbufs[b], gsem.at[b])
        for b in range(4):
            pltpu.make_async_copy(
                wbufs[b], acc_sp.at[didx.at[0]], ssem.at[b]).wait()
        plsc.subcore_barrier()

        sbufs = (bufY, wb2)
        pltpu.async_copy(acc_sp.at[pl.ds(r0, RC)], sbufs[0], gsem.at[0])
        carry = (zero,) * (2 * CH)
        for ch in range(NCH):
            cb = sbufs[ch % 2]
            pltpu.make_async_copy(
                acc_sp.at[pl.ds(r0, RC)], cb, gsem.at[ch % 2]).wait()
            if ch + 1 < NCH:
                pltpu.async_copy(
                    acc_sp.at[pl.ds(r0 + (ch + 1) * RC, RC)],
                    sbufs[(ch + 1) % 2], gsem.at[(ch + 1) % 2])
            nrows = jnp.clip(nreal - ch * RC, 0, RC)

            def stats_body(i, inner, ch=ch, cb=cb):
                isums = list(inner)
                schunk = nslice[pl.ds(ch * RC + i * L, L)]
                for rr in range(L):
                    r = i * L + rr
                    sr = schunk[rr]
                    for c in range(CH):
                        y = (-sr) * cb[r, pl.ds(c * L, L)]
                        isums[c] = isums[c] + y
                        isums[CH + c] = isums[CH + c] + y * y
                return tuple(isums)

            carry = pl.loop(0, nrows // L, init_carry=carry)(stats_body)
        for c in range(CH):
            statv[pl.ds(c * L, L)] = carry[c]
            statv[pl.ds(DH + c * L, L)] = carry[CH + c]
        pltpu.sync_copy(statv, stats_sp.at[tid])
        plsc.subcore_barrier()

        pltpu.sync_copy(stats_sp, statall)
        coeffs = []
        for c in range(CH):
            m = zero
            q = zero
            for t in range(NS):
                m = m + statall[t, pl.ds(c * L, L)]
                q = q + statall[t, pl.ds(DH + c * L, L)]
            m = m * (1.0 / N)
            q = q * (1.0 / N)
            inv = _rsqrt(jnp.maximum(q - m * m, 0.0) + EPS)
            gA = inv * bnwv[k, pl.ds(c * L, L)]
            gB = bnbv[k, pl.ds(c * L, L)] - m * gA
            coeffs.append((gA, gB))
        gvk = gamv[pl.ds(0, L)]
        gk = gvk.at[jnp.full((L,), k + 1, jnp.int32)].get(
            mode="promise_in_bounds")

        pairs = ((bufY, bufH), (wb2, wb3))
        pltpu.async_copy(acc_sp.at[pl.ds(r0, RC)], bufY, gsem.at[0])
        pltpu.async_copy(out_hbm.at[pl.ds(r0, RC)], bufH, gsem.at[2])
        for ch in range(NCH):
            p = ch % 2
            Yb, Hb = pairs[p]
            pltpu.make_async_copy(
                acc_sp.at[pl.ds(r0, RC)], Yb, gsem.at[p]).wait()
            pltpu.make_async_copy(
                out_hbm.at[pl.ds(r0, RC)], Hb, gsem.at[p + 2]).wait()
            if ch + 1 < NCH:
                np_ = (ch + 1) % 2
                Yn, Hn = pairs[np_]
                if ch >= 1:
                    pltpu.make_async_copy(
                        Yn, u_hbm.at[pl.ds(r0, RC)], ssem.at[np_]).wait()
                    pltpu.make_async_copy(
                        Yn, acc_sp.at[pl.ds(r0, RC)], ssem.at[np_ + 2]).wait()
                    pltpu.make_async_copy(
                        Hn, out_hbm.at[pl.ds(r0, RC)], osem.at[np_]).wait()
                pltpu.async_copy(
                    acc_sp.at[pl.ds(r0 + (ch + 1) * RC, RC)], Yn, gsem.at[np_])
                pltpu.async_copy(
                    out_hbm.at[pl.ds(r0 + (ch + 1) * RC, RC)], Hn,
                    gsem.at[np_ + 2])

            @pl.loop(0, RC // L)
            def _(i, ch=ch, Yb=Yb, Hb=Hb):
                schunk = nslice[pl.ds(ch * RC + i * L, L)]
                for rr in range(L):
                    r = i * L + rr
                    sr = schunk[rr]
                    for c in range(CH):
                        gA, gB = coeffs[c]
                        h = ((-sr) * Yb[r, pl.ds(c * L, L)]) * gA + gB
                        Hb[r, pl.ds(c * L, L)] = Hb[r, pl.ds(c * L, L)] + gk * h
                        Yb[r, pl.ds(c * L, L)] = sr * h
            rb = r0 + ch * RC
            pltpu.async_copy(Hb, out_hbm.at[pl.ds(rb, RC)], osem.at[p])
            pltpu.async_copy(Yb, u_hbm.at[pl.ds(rb, RC)], ssem.at[p])
            pltpu.async_copy(Yb, acc_sp.at[pl.ds(rb, RC)], ssem.at[p + 2])
        for ch in (NCH - 2, NCH - 1):
            p = ch % 2
            Yb, Hb = pairs[p]
            pltpu.make_async_copy(
                Yb, u_hbm.at[pl.ds(r0, RC)], ssem.at[p]).wait()
            pltpu.make_async_copy(
                Yb, acc_sp.at[pl.ds(r0, RC)], ssem.at[p + 2]).wait()
            pltpu.make_async_copy(
                Hb, out_hbm.at[pl.ds(r0, RC)], osem.at[p]).wait()
        plsc.subcore_barrier()


def _make_call():
    mesh = plsc.VectorSubcoreMesh(
        core_axis_name="c", subcore_axis_name="s",
        num_cores=NC, num_subcores=NS)
    f32 = jnp.float32
    return pl.kernel(
        _sc_body,
        out_type=(
            jax.ShapeDtypeStruct((NC, NP, DH), f32),
            jax.ShapeDtypeStruct((NC, NP, DH), f32),
        ),
        mesh=mesh,
        compiler_params=pltpu.CompilerParams(use_tc_tiling_on_sc=False),
        scratch_types=[
            pltpu.VMEM_SHARED((NP, DH), f32),
            pltpu.VMEM_SHARED((NP,), f32),
            pltpu.VMEM_SHARED((NP,), f32),
            pltpu.VMEM_SHARED((NS, 2 * DH), f32),
            pltpu.VMEM((NWIN, W), jnp.int32),
            pltpu.VMEM((NWIN, W), jnp.int32),
            pltpu.VMEM((RC, DH), f32),
            pltpu.VMEM((RC, DH), f32),
            pltpu.VMEM((W, DH), f32),
            pltpu.VMEM((W, DH), f32),
            pltpu.VMEM((RT,), f32),
            pltpu.VMEM((RT,), f32),
            pltpu.VMEM((W,), f32),
            pltpu.VMEM((W,), f32),
            pltpu.VMEM((W,), f32),
            pltpu.VMEM((K, DH), f32),
            pltpu.VMEM((K, DH), f32),
            pltpu.VMEM((L,), f32),
            pltpu.VMEM((2 * DH,), f32),
            pltpu.VMEM((NS, 2 * DH), f32),
            pltpu.SemaphoreType.DMA((4,)),
            pltpu.SemaphoreType.DMA((4,)),
            pltpu.SemaphoreType.DMA((2,)),
            pltpu.SemaphoreType.DMA,
        ],
    )


_SC_CALL = _make_call()


def kernel(x, edge_index, temp, bn_weight, bn_bias):
    src = edge_index[0].astype(jnp.int32)
    dst = edge_index[1].astype(jnp.int32)
    npad = EP - E
    ghost = N + (jnp.arange(npad, dtype=jnp.int32) % GH)
    srcw = jnp.concatenate([src, ghost]).reshape(EP // W, W)
    dstw = jnp.concatenate([dst, ghost]).reshape(EP // W, W)
    xs = jnp.stack([x[:, :DH], x[:, DH:]])
    x2 = jnp.zeros((NC, NP, DH), jnp.float32).at[:, :N].set(xs)
    gam16 = jnp.zeros((L,), jnp.float32).at[:K + 1].set(temp / (K + 1))
    bnw2 = jnp.stack([bn_weight[:K, :DH], bn_weight[:K, DH:]])
    bnb2 = jnp.stack([bn_bias[:K, :DH], bn_bias[:K, DH:]])
    out2, _ = _SC_CALL(x2, srcw, dstw, gam16, bnw2, bnb2)
    return jnp.moveaxis(out2[:, :N], 0, 1).reshape(N, D)
